# baseline (speedup 1.0000x reference)
import dataclasses

import jax
import jax.numpy as jnp
from jax.experimental import pallas as pl
from jax.experimental.pallas import tpu as pltpu
from jax.experimental.pallas import tpu_sc as plsc

VOCAB = 1000000
EMB = 32
WIN = 5
HID = 256
OUT = 64
BATCH = 16384
GWIN = 128
NJ = BATCH // GWIN
VCH = 800
NCH = VOCAB // VCH
BB = 2048

_vector_mesh = plsc.VectorSubcoreMesh(
    core_axis_name="core", subcore_axis_name="subcore"
)


def _sc_transpose(cols):
    cp = pltpu.CompilerParams(use_tc_tiling_on_sc=False)
    if "needs_layout_passes" in pltpu.CompilerParams.__dataclass_fields__:
        cp = dataclasses.replace(cp, needs_layout_passes=False)

    @pl.kernel(
        out_type=jax.ShapeDtypeStruct((VOCAB, EMB), jnp.float32),
        mesh=_vector_mesh,
        compiler_params=cp,
    )
    def transpose_kernel(*refs):
        col_refs = refs[:EMB]
        chunk_ids = refs[EMB]
        out_hbm = refs[EMB + 1]

        def body(idx_ref, o_vmem):
            base = jnp.max(idx_ref[0]) * VCH

            def scoped(scratch, sem):
                copies = [
                    pltpu.make_async_copy(
                        col_refs[f].at[pl.ds(base, VCH)], scratch.at[f], sem
                    )
                    for f in range(EMB)
                ]
                for c in copies:
                    c.start()
                for c in copies:
                    c.wait()

                lanes = jax.lax.iota(jnp.int32, 16)

                @pl.loop(0, VCH // 16)
                def _(g):
                    rows = lanes + g * 16
                    for f in range(EMB):
                        dcols = (lanes + f) & (EMB - 1)
                        vals = plsc.load_gather(scratch, [dcols, rows])
                        plsc.store_scatter(o_vmem, [rows, dcols], vals)

            pl.run_scoped(
                scoped,
                pltpu.VMEM((EMB, VCH), jnp.float32),
                pltpu.SemaphoreType.DMA,
            )

        pltpu.emit_pipeline(
            body,
            grid=(NCH,),
            in_specs=[pl.BlockSpec((1, 16), index_map=lambda j: (j, 0))],
            out_specs=[pl.BlockSpec((VCH, EMB), index_map=lambda j: (j, 0))],
            core_axis_name=("core", "subcore"),
            dimension_semantics=(pltpu.PARALLEL,),
        )(chunk_ids, out_hbm)

    cids = jnp.broadcast_to(
        jnp.arange(NCH, dtype=jnp.int32)[:, None], (NCH, 16)
    )
    return transpose_kernel(*cols, cids)


def _sc_gather(tlin, x8):
    @pl.kernel(
        out_type=jax.ShapeDtypeStruct((2, BATCH, 128), jnp.float32),
        mesh=_vector_mesh,
        compiler_params=pltpu.CompilerParams(use_tc_tiling_on_sc=False),
    )
    def gather_kernel(table_hbm, idx_hbm, out_hbm):
        def body(i_vmem, o_vmem):
            pltpu.sync_copy(table_hbm.at[i_vmem.at[0]], o_vmem.at[0])

        pltpu.emit_pipeline(
            body,
            grid=(8, NJ),
            in_specs=[pl.BlockSpec((1, GWIN), index_map=lambda w, j: (w, j))],
            out_specs=[
                pl.BlockSpec(
                    (1, GWIN, EMB),
                    index_map=lambda w, j: (w // 4, j, w % 4),
                )
            ],
            core_axis_name=("core", "subcore"),
            dimension_semantics=(pltpu.PARALLEL, pltpu.PARALLEL),
        )(idx_hbm, out_hbm)

    return gather_kernel(tlin, x8)


def _mlp_body(e_ref, w1a_ref, w1b_ref, b1_ref, w2_ref, b2_ref, o_ref):
    h = jnp.tanh(
        jnp.dot(e_ref[0], w1a_ref[...], preferred_element_type=jnp.float32)
        + jnp.dot(e_ref[1], w1b_ref[...], preferred_element_type=jnp.float32)
        + b1_ref[...]
    )
    o_ref[...] = (
        jnp.dot(h, w2_ref[...], preferred_element_type=jnp.float32) + b2_ref[...]
    )


def _tc_mlp(eg, W1a, W1b, b1, W2, b2):
    return pl.pallas_call(
        _mlp_body,
        grid=(BATCH // BB,),
        in_specs=[
            pl.BlockSpec((2, BB, 128), lambda i: (0, i, 0)),
            pl.BlockSpec((128, HID), lambda i: (0, 0)),
            pl.BlockSpec((128, HID), lambda i: (0, 0)),
            pl.BlockSpec((1, HID), lambda i: (0, 0)),
            pl.BlockSpec((HID, OUT), lambda i: (0, 0)),
            pl.BlockSpec((1, OUT), lambda i: (0, 0)),
        ],
        out_specs=pl.BlockSpec((BB, OUT), lambda i: (i, 0)),
        out_shape=jax.ShapeDtypeStruct((BATCH, OUT), jnp.float32),
    )(eg, W1a, W1b, b1.reshape(1, HID), W2, b2.reshape(1, OUT))


@jax.jit
def kernel(x, table, W1, b1, W2, b2):
    xi = x.astype(jnp.int32)
    cols = [xi[:, w] for w in range(WIN)]
    x8 = jnp.stack(cols + cols[1:4])
    tcols = [table[:, f] for f in range(EMB)]
    tlin = _sc_transpose(tcols)
    eg = _sc_gather(tlin, x8)
    W1a = W1[:128]
    W1b = jnp.zeros((128, HID), jnp.float32).at[: WIN * EMB - 128].set(W1[128:])
    return _tc_mlp(eg, W1a, W1b, b1, W2, b2)


# speedup vs baseline: 2.0354x; 2.0354x over previous
import jax
import jax.numpy as jnp
from jax.experimental import pallas as pl
from jax.experimental.pallas import tpu as pltpu
from jax.experimental.pallas import tpu_sc as plsc

VOCAB = 1000000
EMB = 32
WIN = 5
HID = 256
OUT = 64
BATCH = 16384
GWIN = 128
NJ = BATCH // GWIN
BB = 2048

_vector_mesh = plsc.VectorSubcoreMesh(
    core_axis_name="core", subcore_axis_name="subcore"
)


def _sc_gather(table, x8):
    @pl.kernel(
        out_type=jax.ShapeDtypeStruct((2, BATCH, 128), jnp.float32),
        mesh=_vector_mesh,
        compiler_params=pltpu.CompilerParams(use_tc_tiling_on_sc=False),
    )
    def gather_kernel(table_hbm, idx_hbm, out_hbm):
        def body(i_vmem, o_vmem):
            pltpu.sync_copy(table_hbm.at[i_vmem.at[0]], o_vmem.at[0])

        pltpu.emit_pipeline(
            body,
            grid=(8, NJ),
            in_specs=[pl.BlockSpec((1, GWIN), index_map=lambda w, j: (w, j))],
            out_specs=[
                pl.BlockSpec(
                    (1, GWIN, EMB),
                    index_map=lambda w, j: (w // 4, j, w % 4),
                )
            ],
            core_axis_name=("core", "subcore"),
            dimension_semantics=(pltpu.PARALLEL, pltpu.PARALLEL),
        )(idx_hbm, out_hbm)

    return gather_kernel(table, x8)


def _mlp_body(e_ref, w1a_ref, w1b_ref, b1_ref, w2_ref, b2_ref, o_ref):
    h = jnp.tanh(
        jnp.dot(e_ref[0], w1a_ref[...], preferred_element_type=jnp.float32)
        + jnp.dot(e_ref[1], w1b_ref[...], preferred_element_type=jnp.float32)
        + b1_ref[...]
    )
    o_ref[...] = (
        jnp.dot(h, w2_ref[...], preferred_element_type=jnp.float32) + b2_ref[...]
    )


def _tc_mlp(eg, W1a, W1b, b1, W2, b2):
    return pl.pallas_call(
        _mlp_body,
        grid=(BATCH // BB,),
        in_specs=[
            pl.BlockSpec((2, BB, 128), lambda i: (0, i, 0)),
            pl.BlockSpec((128, HID), lambda i: (0, 0)),
            pl.BlockSpec((128, HID), lambda i: (0, 0)),
            pl.BlockSpec((1, HID), lambda i: (0, 0)),
            pl.BlockSpec((HID, OUT), lambda i: (0, 0)),
            pl.BlockSpec((1, OUT), lambda i: (0, 0)),
        ],
        out_specs=pl.BlockSpec((BB, OUT), lambda i: (i, 0)),
        out_shape=jax.ShapeDtypeStruct((BATCH, OUT), jnp.float32),
    )(eg, W1a, W1b, b1.reshape(1, HID), W2, b2.reshape(1, OUT))


@jax.jit
def kernel(x, table, W1, b1, W2, b2):
    xi = x.astype(jnp.int32)
    cols = [xi[:, w] * 4 for w in range(WIN)]
    x8 = jnp.stack(cols + cols[1:4])
    t4 = jnp.pad(table, ((0, 0), (0, 96))).reshape(4 * VOCAB, EMB)
    eg = _sc_gather(t4, x8)
    W1a = W1[:128]
    W1b = jnp.zeros((128, HID), jnp.float32).at[: WIN * EMB - 128].set(W1[128:])
    return _tc_mlp(eg, W1a, W1b, b1, W2, b2)
